# baseline (speedup 1.0000x reference)
import functools

import jax
import jax.numpy as jnp
from jax import lax
from jax.experimental import pallas as pl
from jax.experimental.pallas import tpu as pltpu
from jax.experimental.pallas import tpu_sc as plsc

_NC = 2
_NS = 16
_NW = _NC * _NS
_CHUNK = 1024
_L = 16

_DNUMS = lax.GatherDimensionNumbers(
    offset_dims=(), collapsed_slice_dims=(0,), start_index_map=(0,))


def _perm(v, pidx):
    return lax.gather(v, pidx[:, None], _DNUMS, slice_sizes=(1,),
                      mode=lax.GatherScatterMode.PROMISE_IN_BOUNDS)


def _allsum(v, lanes):
    for k in (1, 2, 4, 8):
        v = v + _perm(v, lanes ^ k)
    return v


def _rsqrt(v):
    i = lax.bitcast_convert_type(v, jnp.int32)
    i = jnp.int32(0x5F3759DF) - lax.shift_right_logical(i, 1)
    y = lax.bitcast_convert_type(i, jnp.float32)
    half_v = 0.5 * v
    for _ in range(2):
        y = y * (1.5 - half_v * y * y)
    return y


def _sqrt_pos(v):
    vs = jnp.maximum(v, jnp.float32(1e-30))
    return vs * _rsqrt(vs)


def _jacobi_rotate(Am, Vm, p, q):
    r = 3 - p - q
    app, aqq, apq = Am[p][p], Am[q][q], Am[p][q]
    apr, aqr = Am[p][r], Am[q][r]

    apq_zero = apq == 0.0
    apq_safe = jnp.where(apq_zero, jnp.float32(1.0), apq)
    tau = (aqq - app) * 0.5 / apq_safe
    tau = jnp.maximum(jnp.minimum(tau, jnp.float32(1e18)), jnp.float32(-1e18))
    sgn = jnp.where(tau >= 0.0, jnp.float32(1.0), jnp.float32(-1.0))
    t = sgn / (jnp.abs(tau) + _sqrt_pos(1.0 + tau * tau))
    t = jnp.where(apq_zero, jnp.float32(0.0), t)
    c = _rsqrt(1.0 + t * t)
    s = t * c

    Am[p][p] = app - t * apq
    Am[q][q] = aqq + t * apq
    zero = apq * 0.0
    Am[p][q] = zero
    Am[q][p] = zero
    npr = c * apr - s * aqr
    nqr = c * aqr + s * apr
    Am[p][r] = npr
    Am[r][p] = npr
    Am[q][r] = nqr
    Am[r][q] = nqr
    for i in range(3):
        vip, viq = Vm[i][p], Vm[i][q]
        Vm[i][p] = c * vip - s * viq
        Vm[i][q] = s * vip + c * viq


def _sc_encoder(t3, idx, n_clusts, S):
    B = idx.shape[0]
    N = t3.shape[2]
    b_per_w = B // _NW
    n_chunks = b_per_w // _CHUNK
    n_vec = b_per_w // _L
    n_stage = N // _NS
    clusts_per_core = n_clusts // _NC

    mesh = plsc.VectorSubcoreMesh(core_axis_name="c", subcore_axis_name="s")

    @functools.partial(
        pl.kernel,
        mesh=mesh,
        out_type=jax.ShapeDtypeStruct((n_clusts * 16,), jnp.float32),
        scratch_types=[
            pltpu.VMEM_SHARED((N,), jnp.float32),
            pltpu.VMEM_SHARED((N,), jnp.float32),
            pltpu.VMEM_SHARED((N,), jnp.float32),
            pltpu.VMEM_SHARED((_NS * _L,), jnp.float32),
            pltpu.VMEM_SHARED((_NS * _L,), jnp.float32),
            pltpu.VMEM((b_per_w,), jnp.int32),
            pltpu.VMEM((b_per_w,), jnp.float32),
            pltpu.VMEM((b_per_w,), jnp.float32),
            pltpu.VMEM((b_per_w,), jnp.float32),
            pltpu.VMEM((_L,), jnp.float32),
            pltpu.VMEM((_L,), jnp.float32),
            pltpu.VMEM((_L,), jnp.float32),
            pltpu.SemaphoreType.DMA,
        ],
    )
    def enc_kernel(t3_hbm, idx_hbm, out,
                   spx, spy, spz, sh1, sh2, idx_v, bx, by, bz,
                   exbuf, pbuf, rowbuf, sem):
        cid = lax.axis_index("c")
        sid = lax.axis_index("s")
        wid = cid * _NS + sid
        base = wid * b_per_w
        pltpu.sync_copy(idx_hbm.at[pl.ds(base, b_per_w)], idx_v)
        copies = []
        for tab, buf in ((t3_hbm.at[0, 0], bx), (t3_hbm.at[1, 0], by),
                         (t3_hbm.at[2, 0], bz)):
            for j in range(n_chunks):
                sl = pl.ds(j * _CHUNK, _CHUNK)
                copies.append(pltpu.async_copy(tab.at[idx_v.at[sl]],
                                               buf.at[sl], sem))
        for cp in copies:
            cp.wait()

        def p1_body(i, acc):
            sx, sy, sz, sxx, syy, szz, sxy, sxz, syz = acc
            sl = pl.ds(i * _L, _L)
            lx = bx[sl]
            ly = by[sl]
            lz = bz[sl]
            return (sx + lx, sy + ly, sz + lz,
                    sxx + lx * lx, syy + ly * ly, szz + lz * lz,
                    sxy + lx * ly, sxz + lx * lz, syz + ly * lz)

        zeros = jnp.zeros((_L,), jnp.float32)
        acc = lax.fori_loop(0, n_vec, p1_body, (zeros,) * 9)

        lanes = lax.iota(jnp.int32, _L)
        ex = jnp.zeros((_L,), jnp.float32)
        for k in range(9):
            ex = jnp.where(lanes == k, _allsum(acc[k], lanes), ex)
        exbuf[...] = ex
        pltpu.sync_copy(exbuf, sh1.at[pl.ds(sid * _L, _L)])
        plsc.subcore_barrier()
        pltpu.sync_copy(sh1.at[pl.ds((sid ^ 1) * _L, _L)], pbuf)
        tot = ex + pbuf[...]

        def lane(k):
            return _perm(tot, jnp.full((_L,), k, jnp.int32))

        fS = jnp.float32(float(S))
        inv = jnp.float32(1.0 / S)
        cx = lane(0) * inv
        cy = lane(1) * inv
        cz = lane(2) * inv
        axx = lane(3) - fS * cx * cx
        ayy = lane(4) - fS * cy * cy
        azz = lane(5) - fS * cz * cz
        axy = lane(6) - fS * cx * cy
        axz = lane(7) - fS * cx * cz
        ayz = lane(8) - fS * cy * cz

        Am = [[axx, axy, axz], [axy, ayy, ayz], [axz, ayz, azz]]
        one = jnp.ones((_L,), jnp.float32)
        zer = jnp.zeros((_L,), jnp.float32)
        Vm = [[one, zer, zer], [zer, one, zer], [zer, zer, one]]
        for _ in range(4):
            _jacobi_rotate(Am, Vm, 0, 1)
            _jacobi_rotate(Am, Vm, 0, 2)
            _jacobi_rotate(Am, Vm, 1, 2)

        wa, wb, wc = Am[0][0], Am[1][1], Am[2][2]
        w2 = jnp.maximum(jnp.maximum(wa, wb), wc)
        w0 = jnp.minimum(jnp.minimum(wa, wb), wc)
        w1 = wa + wb + wc - w2 - w0
        onev = jnp.float32(1.0)
        zerov = jnp.float32(0.0)
        fa = (jnp.where(wa >= wb, onev, zerov)
              * jnp.where(wa >= wc, onev, zerov))
        fb = (onev - fa) * jnp.where(wb >= wc, onev, zerov)
        fc = onev - fa - fb
        v2x = fa * Vm[0][0] + fb * Vm[0][1] + fc * Vm[0][2]
        v2y = fa * Vm[1][0] + fb * Vm[1][1] + fc * Vm[1][2]
        v2z = fa * Vm[2][0] + fb * Vm[2][1] + fc * Vm[2][2]
        dirwt = 1.0 - w1 / w2
        iw2 = 1.0 / w2

        def p2_body(i, sacc):
            sl = pl.ds(i * _L, _L)
            xc = bx[sl] - cx
            yc = by[sl] - cy
            zc = bz[sl] - cz
            x0 = xc * v2x + yc * v2y + zc * v2z
            rr = xc * xc + yc * yc + zc * zc - x0 * x0
            np0 = _sqrt_pos(rr)
            return sacc + x0 * np0

        sacc = lax.fori_loop(0, n_vec, p2_body, zeros)
        scv = _allsum(sacc, lanes)
        exbuf[...] = scv
        pltpu.sync_copy(exbuf, sh2.at[pl.ds(sid * _L, _L)])
        plsc.subcore_barrier()
        pltpu.sync_copy(sh2.at[pl.ds((sid ^ 1) * _L, _L)], pbuf)
        sc_v = scv + pbuf[...]

        @pl.when(sid % 2 == 0)
        def _():
            flip = jnp.where(sc_v < 0.0, -dirwt, dirwt)
            v0x = flip * v2x
            v0y = flip * v2y
            v0z = flip * v2z
            vals = [cx, cy, cz,
                    axx * iw2, axy * iw2, axz * iw2,
                    axy * iw2, ayy * iw2, ayz * iw2,
                    axz * iw2, ayz * iw2, azz * iw2,
                    v0x, v0y, v0z, jnp.full((_L,), fS, jnp.float32)]
            row = jnp.zeros((_L,), jnp.float32)
            for k, v in enumerate(vals):
                row = jnp.where(lanes == k, v, row)
            rowbuf[...] = row
            cluster = cid * clusts_per_core + sid // 2
            pltpu.sync_copy(rowbuf, out.at[pl.ds(cluster * 16, 16)])

    return enc_kernel(t3, idx)


def kernel(data, clusts):
    n_clusts, S = clusts.shape
    voxels = data[:, 0:3].astype(jnp.float32)
    t3 = voxels.T.reshape(3, 1, -1)
    idx = clusts.reshape(-1).astype(jnp.int32)
    feats = _sc_encoder(t3, idx, n_clusts, S)
    return feats.reshape(n_clusts, 16)


# speedup vs baseline: 1.1107x; 1.1107x over previous
import functools

import jax
import jax.numpy as jnp
from jax import lax
from jax.experimental import pallas as pl
from jax.experimental.pallas import tpu as pltpu
from jax.experimental.pallas import tpu_sc as plsc

_NC = 2
_NS = 16
_NW = _NC * _NS
_CHUNK = 1024
_L = 16

_DNUMS = lax.GatherDimensionNumbers(
    offset_dims=(), collapsed_slice_dims=(0,), start_index_map=(0,))


def _perm(v, pidx):
    return lax.gather(v, pidx[:, None], _DNUMS, slice_sizes=(1,),
                      mode=lax.GatherScatterMode.PROMISE_IN_BOUNDS)


def _allsum(v, lanes):
    for k in (1, 2, 4, 8):
        v = v + _perm(v, lanes ^ k)
    return v


def _rsqrt(v):
    i = lax.bitcast_convert_type(v, jnp.int32)
    i = jnp.int32(0x5F3759DF) - lax.shift_right_logical(i, 1)
    y = lax.bitcast_convert_type(i, jnp.float32)
    half_v = 0.5 * v
    for _ in range(2):
        y = y * (1.5 - half_v * y * y)
    return y


def _sqrt_pos(v):
    vs = jnp.maximum(v, jnp.float32(1e-30))
    return vs * _rsqrt(vs)


def _jacobi_rotate(Am, Vm, p, q):
    r = 3 - p - q
    app, aqq, apq = Am[p][p], Am[q][q], Am[p][q]
    apr, aqr = Am[p][r], Am[q][r]

    apq_zero = apq == 0.0
    apq_safe = jnp.where(apq_zero, jnp.float32(1.0), apq)
    tau = (aqq - app) * 0.5 / apq_safe
    tau = jnp.maximum(jnp.minimum(tau, jnp.float32(1e18)), jnp.float32(-1e18))
    sgn = jnp.where(tau >= 0.0, jnp.float32(1.0), jnp.float32(-1.0))
    t = sgn / (jnp.abs(tau) + _sqrt_pos(1.0 + tau * tau))
    t = jnp.where(apq_zero, jnp.float32(0.0), t)
    c = _rsqrt(1.0 + t * t)
    s = t * c

    Am[p][p] = app - t * apq
    Am[q][q] = aqq + t * apq
    zero = apq * 0.0
    Am[p][q] = zero
    Am[q][p] = zero
    npr = c * apr - s * aqr
    nqr = c * aqr + s * apr
    Am[p][r] = npr
    Am[r][p] = npr
    Am[q][r] = nqr
    Am[r][q] = nqr
    for i in range(3):
        vip, viq = Vm[i][p], Vm[i][q]
        Vm[i][p] = c * vip - s * viq
        Vm[i][q] = s * vip + c * viq


def _sc_encoder(t3, idx, n_clusts, S):
    B = idx.shape[0]
    N = t3.shape[2]
    b_per_w = B // _NW
    n_chunks = b_per_w // _CHUNK
    n_vec = b_per_w // _L
    n_stage = N // _NS
    clusts_per_core = n_clusts // _NC

    mesh = plsc.VectorSubcoreMesh(core_axis_name="c", subcore_axis_name="s")

    @functools.partial(
        pl.kernel,
        mesh=mesh,
        out_type=jax.ShapeDtypeStruct((n_clusts * 16,), jnp.float32),
        scratch_types=[
            pltpu.VMEM_SHARED((N,), jnp.float32),
            pltpu.VMEM_SHARED((N,), jnp.float32),
            pltpu.VMEM_SHARED((N,), jnp.float32),
            pltpu.VMEM_SHARED((_NS * _L,), jnp.float32),
            pltpu.VMEM_SHARED((_NS * _L,), jnp.float32),
            pltpu.VMEM((b_per_w,), jnp.int32),
            pltpu.VMEM((b_per_w,), jnp.float32),
            pltpu.VMEM((b_per_w,), jnp.float32),
            pltpu.VMEM((b_per_w,), jnp.float32),
            pltpu.VMEM((_L,), jnp.float32),
            pltpu.VMEM((_L,), jnp.float32),
            pltpu.VMEM((_L,), jnp.float32),
            pltpu.SemaphoreType.DMA,
        ],
    )
    def enc_kernel(t3_hbm, idx_hbm, out,
                   spx, spy, spz, sh1, sh2, idx_v, bx, by, bz,
                   exbuf, pbuf, rowbuf, sem):
        cid = lax.axis_index("c")
        sid = lax.axis_index("s")
        wid = cid * _NS + sid
        base = wid * b_per_w
        st = sid * n_stage
        stage = [
            pltpu.async_copy(t3_hbm.at[0, 0, pl.ds(st, n_stage)],
                             spx.at[pl.ds(st, n_stage)], sem),
            pltpu.async_copy(t3_hbm.at[1, 0, pl.ds(st, n_stage)],
                             spy.at[pl.ds(st, n_stage)], sem),
            pltpu.async_copy(t3_hbm.at[2, 0, pl.ds(st, n_stage)],
                             spz.at[pl.ds(st, n_stage)], sem),
        ]
        pltpu.sync_copy(idx_hbm.at[pl.ds(base, b_per_w)], idx_v)
        for cp in stage:
            cp.wait()
        plsc.subcore_barrier()
        copies = []
        for tab, buf in ((spx, bx), (spy, by), (spz, bz)):
            per = []
            for j in range(n_chunks):
                sl = pl.ds(j * _CHUNK, _CHUNK)
                per.append(pltpu.async_copy(tab.at[idx_v.at[sl]],
                                            buf.at[sl], sem))
            copies.append(per)

        zeros = jnp.zeros((_L,), jnp.float32)

        def pA(i, acc):
            sx, sxx = acc
            lx = bx[pl.ds(i * _L, _L)]
            return (sx + lx, sxx + lx * lx)

        def pB(i, acc):
            sy, syy, sxy = acc
            sl = pl.ds(i * _L, _L)
            lx = bx[sl]
            ly = by[sl]
            return (sy + ly, syy + ly * ly, sxy + lx * ly)

        def pC(i, acc):
            sz, szz, sxz, syz = acc
            sl = pl.ds(i * _L, _L)
            lx = bx[sl]
            ly = by[sl]
            lz = bz[sl]
            return (sz + lz, szz + lz * lz, sxz + lx * lz, syz + ly * lz)

        for cp in copies[0]:
            cp.wait()
        accA = lax.fori_loop(0, n_vec, pA, (zeros,) * 2)
        for cp in copies[1]:
            cp.wait()
        accB = lax.fori_loop(0, n_vec, pB, (zeros,) * 3)
        for cp in copies[2]:
            cp.wait()
        accC = lax.fori_loop(0, n_vec, pC, (zeros,) * 4)
        acc = (accA[0], accB[0], accC[0], accA[1], accB[1], accC[1],
               accB[2], accC[2], accC[3])

        lanes = lax.iota(jnp.int32, _L)
        ex = jnp.zeros((_L,), jnp.float32)
        for k in range(9):
            ex = jnp.where(lanes == k, _allsum(acc[k], lanes), ex)
        exbuf[...] = ex
        pltpu.sync_copy(exbuf, sh1.at[pl.ds(sid * _L, _L)])
        plsc.subcore_barrier()
        pltpu.sync_copy(sh1.at[pl.ds((sid ^ 1) * _L, _L)], pbuf)
        tot = ex + pbuf[...]

        def lane(k):
            return _perm(tot, jnp.full((_L,), k, jnp.int32))

        fS = jnp.float32(float(S))
        inv = jnp.float32(1.0 / S)
        cx = lane(0) * inv
        cy = lane(1) * inv
        cz = lane(2) * inv
        axx = lane(3) - fS * cx * cx
        ayy = lane(4) - fS * cy * cy
        azz = lane(5) - fS * cz * cz
        axy = lane(6) - fS * cx * cy
        axz = lane(7) - fS * cx * cz
        ayz = lane(8) - fS * cy * cz

        Am = [[axx, axy, axz], [axy, ayy, ayz], [axz, ayz, azz]]
        one = jnp.ones((_L,), jnp.float32)
        zer = jnp.zeros((_L,), jnp.float32)
        Vm = [[one, zer, zer], [zer, one, zer], [zer, zer, one]]
        for _ in range(3):
            _jacobi_rotate(Am, Vm, 0, 1)
            _jacobi_rotate(Am, Vm, 0, 2)
            _jacobi_rotate(Am, Vm, 1, 2)

        wa, wb, wc = Am[0][0], Am[1][1], Am[2][2]
        w2 = jnp.maximum(jnp.maximum(wa, wb), wc)
        w0 = jnp.minimum(jnp.minimum(wa, wb), wc)
        w1 = wa + wb + wc - w2 - w0
        onev = jnp.float32(1.0)
        zerov = jnp.float32(0.0)
        fa = (jnp.where(wa >= wb, onev, zerov)
              * jnp.where(wa >= wc, onev, zerov))
        fb = (onev - fa) * jnp.where(wb >= wc, onev, zerov)
        fc = onev - fa - fb
        v2x = fa * Vm[0][0] + fb * Vm[0][1] + fc * Vm[0][2]
        v2y = fa * Vm[1][0] + fb * Vm[1][1] + fc * Vm[1][2]
        v2z = fa * Vm[2][0] + fb * Vm[2][1] + fc * Vm[2][2]
        dirwt = 1.0 - w1 / w2
        iw2 = 1.0 / w2

        def p2_body(i, sacc):
            sl = pl.ds(i * _L, _L)
            xc = bx[sl] - cx
            yc = by[sl] - cy
            zc = bz[sl] - cz
            x0 = xc * v2x + yc * v2y + zc * v2z
            rr = xc * xc + yc * yc + zc * zc - x0 * x0
            np0 = _sqrt_pos(rr)
            return sacc + x0 * np0

        sacc = lax.fori_loop(0, n_vec, p2_body, zeros)
        scv = _allsum(sacc, lanes)
        exbuf[...] = scv
        pltpu.sync_copy(exbuf, sh2.at[pl.ds(sid * _L, _L)])
        plsc.subcore_barrier()
        pltpu.sync_copy(sh2.at[pl.ds((sid ^ 1) * _L, _L)], pbuf)
        sc_v = scv + pbuf[...]

        @pl.when(sid % 2 == 0)
        def _():
            flip = jnp.where(sc_v < 0.0, -dirwt, dirwt)
            v0x = flip * v2x
            v0y = flip * v2y
            v0z = flip * v2z
            vals = [cx, cy, cz,
                    axx * iw2, axy * iw2, axz * iw2,
                    axy * iw2, ayy * iw2, ayz * iw2,
                    axz * iw2, ayz * iw2, azz * iw2,
                    v0x, v0y, v0z, jnp.full((_L,), fS, jnp.float32)]
            row = jnp.zeros((_L,), jnp.float32)
            for k, v in enumerate(vals):
                row = jnp.where(lanes == k, v, row)
            rowbuf[...] = row
            cluster = cid * clusts_per_core + sid // 2
            pltpu.sync_copy(rowbuf, out.at[pl.ds(cluster * 16, 16)])

    return enc_kernel(t3, idx)


def kernel(data, clusts):
    n_clusts, S = clusts.shape
    voxels = data[:, 0:3].astype(jnp.float32)
    t3 = voxels.T.reshape(3, 1, -1)
    idx = clusts.reshape(-1).astype(jnp.int32)
    feats = _sc_encoder(t3, idx, n_clusts, S)
    return feats.reshape(n_clusts, 16)
